# Initial kernel scaffold; baseline (speedup 1.0000x reference)
import jax
import jax.numpy as jnp
from jax import lax
from jax.experimental import pallas as pl
from jax.experimental.pallas import tpu as pltpu
from jax.experimental.pallas import tpu_sc as plsc

NC = 2
NS = 16
L = 16

V = 100_000
V_PAD = 102_400
F = 200_000
F_PAD = 204_800
NF_TILE = F_PAD // NS
CHUNK = 512
NCHUNK = NF_TILE // CHUNK
SUB = CHUNK // 128
NV_TILE = V_PAD // NS


def _iota16():
    return lax.iota(jnp.int32, L)


def _full16(v):
    return jnp.full((L,), v, dtype=jnp.int32)


def _rsqrt(s):
    i = plsc.bitcast(s, jnp.int32)
    i = 0x5F3759DF - lax.shift_right_arithmetic(i, 1)
    y = plsc.bitcast(i, jnp.float32)
    h = 0.5 * s
    for _ in range(3):
        y = y * (1.5 - h * y * y)
    return y


def _sc_body(table, faces_off, faces_raw, out, areas,
             idxg0, idxg1, idxg2, idxs0, idxs1, idxs2,
             g0, g1, g2, nrm, ar0, ar1, zbuf, fin, acc, sem):
    c = lax.axis_index("c")
    s = lax.axis_index("s")
    tile_face0 = s * NF_TILE
    tile_v0 = s * NV_TILE
    iota = _iota16()

    def zb(i, _):
        rows = 2 * i + lax.shift_right_logical(iota, 3)
        cols = lax.bitwise_and(iota, _full16(7))
        plsc.store_scatter(zbuf, [rows, cols], jnp.zeros((L,), jnp.float32))
        return _
    lax.fori_loop(0, 32, zb, None)

    for u in range(CHUNK // 64):
        pltpu.sync_copy(zbuf, nrm.at[pl.ds(u * 64, 64)])

    def za(i, _):
        pltpu.sync_copy(zbuf, acc.at[pl.ds(tile_v0 + i * 64, 64)])
        return _
    lax.fori_loop(0, NV_TILE // 64, za, None)
    plsc.subcore_barrier()

    def chunk_body(j, _):
        rb = (tile_face0 + j * CHUNK) // 128
        fb = tile_face0 + j * CHUNK
        pltpu.sync_copy(faces_off.at[c, 0, pl.ds(rb, SUB)], idxg0)
        pltpu.sync_copy(faces_off.at[c, 1, pl.ds(rb, SUB)], idxg1)
        pltpu.sync_copy(faces_off.at[c, 2, pl.ds(rb, SUB)], idxg2)
        pltpu.sync_copy(faces_raw.at[0, pl.ds(rb, SUB)], idxs0)
        pltpu.sync_copy(faces_raw.at[1, pl.ds(rb, SUB)], idxs1)
        pltpu.sync_copy(faces_raw.at[2, pl.ds(rb, SUB)], idxs2)
        descs = []
        for u in range(SUB):
            descs.append(pltpu.async_copy(
                table.at[idxg0.at[u]], g0.at[pl.ds(u * 128, 128)], sem))
            descs.append(pltpu.async_copy(
                table.at[idxg1.at[u]], g1.at[pl.ds(u * 128, 128)], sem))
            descs.append(pltpu.async_copy(
                table.at[idxg2.at[u]], g2.at[pl.ds(u * 128, 128)], sem))
        for d in descs:
            d.wait()

        def step(i, _):
            rows = i * L + iota
            for b in (0, 1):
                o = 4 * b
                ax = plsc.load_gather(g0, [rows, _full16(o)])
                ay = plsc.load_gather(g0, [rows, _full16(o + 1)])
                az = plsc.load_gather(g0, [rows, _full16(o + 2)])
                bx = plsc.load_gather(g1, [rows, _full16(o)])
                by = plsc.load_gather(g1, [rows, _full16(o + 1)])
                bz = plsc.load_gather(g1, [rows, _full16(o + 2)])
                cx = plsc.load_gather(g2, [rows, _full16(o)])
                cy = plsc.load_gather(g2, [rows, _full16(o + 1)])
                cz = plsc.load_gather(g2, [rows, _full16(o + 2)])
                e1x, e1y, e1z = bx - ax, by - ay, bz - az
                e2x, e2y, e2z = cx - bx, cy - by, cz - bz
                nx = e1y * e2z - e1z * e2y
                ny = e1z * e2x - e1x * e2z
                nz = e1x * e2y - e1y * e2x
                plsc.store_scatter(nrm, [rows, _full16(o)], nx)
                plsc.store_scatter(nrm, [rows, _full16(o + 1)], ny)
                plsc.store_scatter(nrm, [rows, _full16(o + 2)], nz)
                sq = nx * nx + ny * ny + nz * nz
                area = 0.5 * sq * _rsqrt(sq)
                ar = ar0 if b == 0 else ar1
                ar[pl.ds(i * L, L)] = area
            return _
        lax.fori_loop(0, CHUNK // L, step, None)

        for u in range(SUB):
            sl = pl.ds(u * 128, 128)
            pltpu.sync_copy(nrm.at[sl], acc.at[idxs0.at[u]], add=True)
            pltpu.sync_copy(nrm.at[sl], acc.at[idxs1.at[u]], add=True)
            pltpu.sync_copy(nrm.at[sl], acc.at[idxs2.at[u]], add=True)

        pltpu.sync_copy(ar0, areas.at[c, 0, pl.ds(fb, CHUNK)])
        pltpu.sync_copy(ar1, areas.at[c, 1, pl.ds(fb, CHUNK)])
        return _
    lax.fori_loop(0, NCHUNK, chunk_body, None)

    plsc.subcore_barrier()

    pltpu.sync_copy(acc.at[pl.ds(tile_v0, NV_TILE)], fin)

    def fstep(i, _):
        rows = i * L + iota
        for b in (0, 1):
            o = 4 * b
            x = plsc.load_gather(fin, [rows, _full16(o)])
            y = plsc.load_gather(fin, [rows, _full16(o + 1)])
            z = plsc.load_gather(fin, [rows, _full16(o + 2)])
            sq = x * x + y * y + z * z
            r = jnp.where(sq >= 1e-12, _rsqrt(sq), 1e6)
            plsc.store_scatter(fin, [rows, _full16(o)], x * r)
            plsc.store_scatter(fin, [rows, _full16(o + 1)], y * r)
            plsc.store_scatter(fin, [rows, _full16(o + 2)], z * r)
        return _
    lax.fori_loop(0, NV_TILE // L, fstep, None)

    pltpu.sync_copy(fin, out.at[c, pl.ds(tile_v0, NV_TILE)])


@jax.jit
def kernel(vertices, faces):
    faces = jnp.squeeze(faces)
    v4 = jnp.pad(vertices, ((0, 0), (0, V_PAD - V), (0, 1)))
    table = (v4.reshape(2, 2, V_PAD, 4)
             .transpose(0, 2, 1, 3)
             .reshape(2 * V_PAD, 8))
    ft = jnp.pad(faces.T, ((0, 0), (0, F_PAD - F)))
    faces_raw = ft.reshape(3, F_PAD // 128, 128)
    offs = (jnp.arange(NC, dtype=jnp.int32) * V_PAD)[:, None, None, None]
    faces_off = faces_raw[None] + offs

    mesh = plsc.VectorSubcoreMesh(core_axis_name="c", subcore_axis_name="s")
    run = pl.kernel(
        _sc_body,
        out_type=(
            jax.ShapeDtypeStruct((NC, V_PAD, 8), jnp.float32),
            jax.ShapeDtypeStruct((NC, 2, F_PAD), jnp.float32),
        ),
        mesh=mesh,
        scratch_types=(
            pltpu.VMEM((SUB, 128), jnp.int32),
            pltpu.VMEM((SUB, 128), jnp.int32),
            pltpu.VMEM((SUB, 128), jnp.int32),
            pltpu.VMEM((SUB, 128), jnp.int32),
            pltpu.VMEM((SUB, 128), jnp.int32),
            pltpu.VMEM((SUB, 128), jnp.int32),
            pltpu.VMEM((CHUNK, 8), jnp.float32),
            pltpu.VMEM((CHUNK, 8), jnp.float32),
            pltpu.VMEM((CHUNK, 8), jnp.float32),
            pltpu.VMEM((CHUNK, 8), jnp.float32),
            pltpu.VMEM((CHUNK,), jnp.float32),
            pltpu.VMEM((CHUNK,), jnp.float32),
            pltpu.VMEM((64, 8), jnp.float32),
            pltpu.VMEM((NV_TILE, 8), jnp.float32),
            pltpu.VMEM_SHARED((V_PAD, 8), jnp.float32),
            pltpu.SemaphoreType.DMA,
        ),
    )
    out, areas = run(table, faces_off, faces_raw)

    vectors = (out[:, :V, :].reshape(2, V, 2, 4)
               .transpose(0, 2, 1, 3)
               .reshape(4, V, 4)[..., :3])
    areas_out = areas.reshape(4, F_PAD)[:, :F]
    return (vectors, areas_out)


# speedup vs baseline: 82.8469x; 82.8469x over previous
import jax
import jax.numpy as jnp
from jax import lax
from jax.experimental import pallas as pl
from jax.experimental.pallas import tpu as pltpu
from jax.experimental.pallas import tpu_sc as plsc

NC = 2
NS = 16
L = 16

V = 100_000
V_PAD = 102_400
F = 200_000
F_PAD = 204_800
NF_TILE = F_PAD // NS
CHUNK = 512
NCHUNK = NF_TILE // CHUNK
SUB = CHUNK // 128
NV_TILE = V_PAD // NS


def _iota16():
    return lax.iota(jnp.int32, L)


def _full16(v):
    return jnp.full((L,), v, dtype=jnp.int32)


def _rsqrt(s):
    i = plsc.bitcast(s, jnp.int32)
    i = 0x5F3759DF - lax.shift_right_arithmetic(i, 1)
    y = plsc.bitcast(i, jnp.float32)
    h = 0.5 * s
    for _ in range(3):
        y = y * (1.5 - h * y * y)
    return y


def _sc_body(table, faces_off, faces_raw, out, areas,
             idxg0, idxg1, idxg2, idxs0, idxs1, idxs2,
             g0, g1, g2, nrm, ar0, ar1, zbuf, fin, acc, sem):
    c = lax.axis_index("c")
    s = lax.axis_index("s")
    tile_face0 = s * NF_TILE
    tile_v0 = s * NV_TILE
    iota = _iota16()

    def zb(i, _):
        rows = 2 * i + lax.shift_right_logical(iota, 3)
        cols = lax.bitwise_and(iota, _full16(7))
        plsc.store_scatter(zbuf, [rows, cols], jnp.zeros((L,), jnp.float32))
        return _
    lax.fori_loop(0, 32, zb, None)

    def zn(i, _):
        rows = i * L + iota
        zv = jnp.zeros((L,), jnp.float32)
        plsc.store_scatter(nrm, [rows, _full16(3)], zv)
        plsc.store_scatter(nrm, [rows, _full16(7)], zv)
        return _
    lax.fori_loop(0, CHUNK // L, zn, None)

    def za(i, _):
        pltpu.sync_copy(zbuf, acc.at[pl.ds(tile_v0 + i * 64, 64)])
        return _
    lax.fori_loop(0, NV_TILE // 64, za, None)
    plsc.subcore_barrier()

    def chunk_body(j, _):
        rb = (tile_face0 + j * CHUNK) // 128
        fb = tile_face0 + j * CHUNK
        pltpu.sync_copy(faces_off.at[c, 0, pl.ds(rb, SUB)], idxg0)
        pltpu.sync_copy(faces_off.at[c, 1, pl.ds(rb, SUB)], idxg1)
        pltpu.sync_copy(faces_off.at[c, 2, pl.ds(rb, SUB)], idxg2)
        pltpu.sync_copy(faces_raw.at[0, pl.ds(rb, SUB)], idxs0)
        pltpu.sync_copy(faces_raw.at[1, pl.ds(rb, SUB)], idxs1)
        pltpu.sync_copy(faces_raw.at[2, pl.ds(rb, SUB)], idxs2)
        descs = []
        for u in range(SUB):
            descs.append(pltpu.async_copy(
                table.at[idxg0.at[u]], g0.at[pl.ds(u * 128, 128)], sem))
            descs.append(pltpu.async_copy(
                table.at[idxg1.at[u]], g1.at[pl.ds(u * 128, 128)], sem))
            descs.append(pltpu.async_copy(
                table.at[idxg2.at[u]], g2.at[pl.ds(u * 128, 128)], sem))
        for d in descs:
            d.wait()

        def step(i, _):
            rows = i * L + iota
            for b in (0, 1):
                o = 4 * b
                ax = plsc.load_gather(g0, [rows, _full16(o)])
                ay = plsc.load_gather(g0, [rows, _full16(o + 1)])
                az = plsc.load_gather(g0, [rows, _full16(o + 2)])
                bx = plsc.load_gather(g1, [rows, _full16(o)])
                by = plsc.load_gather(g1, [rows, _full16(o + 1)])
                bz = plsc.load_gather(g1, [rows, _full16(o + 2)])
                cx = plsc.load_gather(g2, [rows, _full16(o)])
                cy = plsc.load_gather(g2, [rows, _full16(o + 1)])
                cz = plsc.load_gather(g2, [rows, _full16(o + 2)])
                e1x, e1y, e1z = bx - ax, by - ay, bz - az
                e2x, e2y, e2z = cx - bx, cy - by, cz - bz
                nx = e1y * e2z - e1z * e2y
                ny = e1z * e2x - e1x * e2z
                nz = e1x * e2y - e1y * e2x
                plsc.store_scatter(nrm, [rows, _full16(o)], nx)
                plsc.store_scatter(nrm, [rows, _full16(o + 1)], ny)
                plsc.store_scatter(nrm, [rows, _full16(o + 2)], nz)
                sq = nx * nx + ny * ny + nz * nz
                area = 0.5 * sq * _rsqrt(sq)
                ar = ar0 if b == 0 else ar1
                ar[pl.ds(i * L, L)] = area
            return _
        lax.fori_loop(0, CHUNK // L, step, None)

        for u in range(SUB):
            sl = pl.ds(u * 128, 128)
            pltpu.sync_copy(nrm.at[sl], acc.at[idxs0.at[u]], add=True)
            pltpu.sync_copy(nrm.at[sl], acc.at[idxs1.at[u]], add=True)
            pltpu.sync_copy(nrm.at[sl], acc.at[idxs2.at[u]], add=True)

        pltpu.sync_copy(ar0, areas.at[c, 0, pl.ds(fb, CHUNK)])
        pltpu.sync_copy(ar1, areas.at[c, 1, pl.ds(fb, CHUNK)])
        return _
    lax.fori_loop(0, NCHUNK, chunk_body, None)

    plsc.subcore_barrier()

    pltpu.sync_copy(acc.at[pl.ds(tile_v0, NV_TILE)], fin)

    def fstep(i, _):
        rows = i * L + iota
        for b in (0, 1):
            o = 4 * b
            x = plsc.load_gather(fin, [rows, _full16(o)])
            y = plsc.load_gather(fin, [rows, _full16(o + 1)])
            z = plsc.load_gather(fin, [rows, _full16(o + 2)])
            sq = x * x + y * y + z * z
            r = jnp.where(sq >= 1e-12, _rsqrt(sq), 1e6)
            plsc.store_scatter(fin, [rows, _full16(o)], x * r)
            plsc.store_scatter(fin, [rows, _full16(o + 1)], y * r)
            plsc.store_scatter(fin, [rows, _full16(o + 2)], z * r)
        return _
    lax.fori_loop(0, NV_TILE // L, fstep, None)

    pltpu.sync_copy(fin, out.at[c, pl.ds(tile_v0, NV_TILE)])


@jax.jit
def kernel(vertices, faces):
    faces = jnp.squeeze(faces)
    v4 = jnp.pad(vertices, ((0, 0), (0, V_PAD - V), (0, 1)))
    table = (v4.reshape(2, 2, V_PAD, 4)
             .transpose(0, 2, 1, 3)
             .reshape(2 * V_PAD, 8))
    ft = jnp.pad(faces.T, ((0, 0), (0, F_PAD - F)))
    faces_raw = ft.reshape(3, F_PAD // 128, 128)
    offs = (jnp.arange(NC, dtype=jnp.int32) * V_PAD)[:, None, None, None]
    faces_off = faces_raw[None] + offs

    mesh = plsc.VectorSubcoreMesh(core_axis_name="c", subcore_axis_name="s")
    run = pl.kernel(
        _sc_body,
        out_type=(
            jax.ShapeDtypeStruct((NC, V_PAD, 8), jnp.float32),
            jax.ShapeDtypeStruct((NC, 2, F_PAD), jnp.float32),
        ),
        mesh=mesh,
        compiler_params=pltpu.CompilerParams(
            use_tc_tiling_on_sc=False, needs_layout_passes=False),
        scratch_types=(
            pltpu.VMEM((SUB, 128), jnp.int32),
            pltpu.VMEM((SUB, 128), jnp.int32),
            pltpu.VMEM((SUB, 128), jnp.int32),
            pltpu.VMEM((SUB, 128), jnp.int32),
            pltpu.VMEM((SUB, 128), jnp.int32),
            pltpu.VMEM((SUB, 128), jnp.int32),
            pltpu.VMEM((CHUNK, 8), jnp.float32),
            pltpu.VMEM((CHUNK, 8), jnp.float32),
            pltpu.VMEM((CHUNK, 8), jnp.float32),
            pltpu.VMEM((CHUNK, 8), jnp.float32),
            pltpu.VMEM((CHUNK,), jnp.float32),
            pltpu.VMEM((CHUNK,), jnp.float32),
            pltpu.VMEM((64, 8), jnp.float32),
            pltpu.VMEM((NV_TILE, 8), jnp.float32),
            pltpu.VMEM_SHARED((V_PAD, 8), jnp.float32),
            pltpu.SemaphoreType.DMA,
        ),
    )
    out, areas = run(table, faces_off, faces_raw)

    vectors = (out[:, :V, :].reshape(2, V, 2, 4)
               .transpose(0, 2, 1, 3)
               .reshape(4, V, 4)[..., :3])
    areas_out = areas.reshape(4, F_PAD)[:, :F]
    return (vectors, areas_out)
